# baseline (speedup 1.0000x reference)
import functools

import jax
import jax.numpy as jnp
from jax import lax
from jax.experimental import pallas as pl
from jax.experimental.pallas import tpu as pltpu
from jax.experimental.pallas import tpu_sc as plsc

N = 50000
D = 32
E = 1600000

TILES = 16
NPAD = 50048
RSLICE = NPAD // TILES
QCH = RSLICE // 8

RPT = 784
ROWS = TILES * RPT
EPAD = ROWS * 128
CH = 7
NBLK = RPT // CH

_MESH = plsc.VectorSubcoreMesh(
    core_axis_name="c", subcore_axis_name="s", num_cores=2, num_subcores=TILES
)


@functools.partial(
    pl.kernel,
    out_type=(
        jax.ShapeDtypeStruct((NPAD,), jnp.float32),
        jax.ShapeDtypeStruct((NPAD,), jnp.float32),
    ),
    mesh=_MESH,
    scratch_types=[
        pltpu.VMEM((CH, 128), jnp.int32),
        pltpu.VMEM((128,), jnp.float32),
        pltpu.VMEM((RSLICE,), jnp.float32),
        pltpu.VMEM_SHARED((NPAD,), jnp.float32),
    ],
    compiler_params=pltpu.CompilerParams(use_tc_tiling_on_sc=False),
)
def _deg_kernel(src_hbm, dst_hbm, zeros1_hbm, ones_hbm,
                degu_hbm, degi_hbm, idx_v, ones_v, stage_v, acc):
    cid = lax.axis_index("c")
    sid = lax.axis_index("s")
    o0 = sid * RSLICE
    pltpu.sync_copy(ones_hbm, ones_v)
    pltpu.sync_copy(zeros1_hbm.at[pl.ds(o0, RSLICE)], stage_v)
    pltpu.sync_copy(stage_v, acc.at[pl.ds(o0, RSLICE)])
    plsc.subcore_barrier()

    def run(idx_hbm):
        def blk(b, carry):
            r0 = sid * RPT + b * CH
            pltpu.sync_copy(idx_hbm.at[pl.ds(r0, CH)], idx_v)
            for j in range(CH):
                pltpu.sync_copy(ones_v, acc.at[idx_v.at[j]], add=True)
            return carry
        lax.fori_loop(0, NBLK, blk, 0)

    @pl.when(cid == 0)
    def _():
        run(src_hbm)

    @pl.when(cid == 1)
    def _():
        run(dst_hbm)

    plsc.subcore_barrier()

    def wout(out_hbm):
        pltpu.sync_copy(acc.at[pl.ds(o0, RSLICE)], stage_v)
        pltpu.sync_copy(stage_v, out_hbm.at[pl.ds(o0, RSLICE)])

    @pl.when(cid == 0)
    def _():
        wout(degu_hbm)

    @pl.when(cid == 1)
    def _():
        wout(degi_hbm)


_WCH = (512, 512, 512, 512, 512, 512, 56)


@functools.partial(
    pl.kernel,
    out_type=(
        jax.ShapeDtypeStruct((NPAD, D), jnp.float32),
        jax.ShapeDtypeStruct((NPAD, D), jnp.float32),
    ),
    mesh=_MESH,
    scratch_types=[
        pltpu.VMEM((4, 128), jnp.int32),
        pltpu.VMEM((4, 128), jnp.int32),
        pltpu.VMEM((512, D), jnp.float32),
        pltpu.VMEM_SHARED((NPAD, D), jnp.float32),
        pltpu.SemaphoreType.DMA,
        pltpu.SemaphoreType.DMA,
        pltpu.SemaphoreType.DMA,
        pltpu.SemaphoreType.DMA,
    ],
    compiler_params=pltpu.CompilerParams(use_tc_tiling_on_sc=False),
)
def _layer_kernel(yu_hbm, yi_hbm, src_hbm, dst_hbm, zeros2_hbm,
                  si_hbm, su_hbm, gidx_v, sidx_v, rows_v, acc,
                  gsem_a, gsem_b, ssem_a, ssem_b):
    cid = lax.axis_index("c")
    sid = lax.axis_index("s")
    o0 = sid * RSLICE
    q0 = o0
    for w in _WCH:
        pltpu.sync_copy(zeros2_hbm.at[pl.ds(q0, w)], rows_v.at[pl.ds(0, w)])
        pltpu.sync_copy(rows_v.at[pl.ds(0, w)], acc.at[pl.ds(q0, w)])
        q0 += w
    plsc.subcore_barrier()

    def run(tab_hbm, g_hbm, s_hbm):
        ssem = (ssem_a, ssem_a, ssem_b, ssem_b)

        def scatter_start(j):
            pltpu.async_copy(rows_v.at[pl.ds(j * 128, 128)],
                             acc.at[sidx_v.at[j]], ssem[j], add=True)

        def scatter_wait(j):
            pltpu.make_async_copy(rows_v.at[pl.ds(j * 128, 128)],
                                  acc.at[sidx_v.at[j]], ssem[j]).wait()

        def blk(b, carry):
            r0 = sid * RPT + b * 4
            @pl.when(b > 0)
            def _():
                scatter_wait(0)
                scatter_wait(1)
            pltpu.sync_copy(g_hbm.at[pl.ds(r0, 2)], gidx_v.at[pl.ds(0, 2)])
            pltpu.sync_copy(s_hbm.at[pl.ds(r0, 2)], sidx_v.at[pl.ds(0, 2)])
            ga0 = pltpu.async_copy(tab_hbm.at[gidx_v.at[0]],
                                   rows_v.at[pl.ds(0, 128)], gsem_a)
            ga1 = pltpu.async_copy(tab_hbm.at[gidx_v.at[1]],
                                   rows_v.at[pl.ds(128, 128)], gsem_a)
            @pl.when(b > 0)
            def _():
                scatter_wait(2)
                scatter_wait(3)
            pltpu.sync_copy(g_hbm.at[pl.ds(r0 + 2, 2)], gidx_v.at[pl.ds(2, 2)])
            pltpu.sync_copy(s_hbm.at[pl.ds(r0 + 2, 2)], sidx_v.at[pl.ds(2, 2)])
            gb0 = pltpu.async_copy(tab_hbm.at[gidx_v.at[2]],
                                   rows_v.at[pl.ds(256, 128)], gsem_b)
            gb1 = pltpu.async_copy(tab_hbm.at[gidx_v.at[3]],
                                   rows_v.at[pl.ds(384, 128)], gsem_b)
            ga0.wait()
            scatter_start(0)
            ga1.wait()
            scatter_start(1)
            gb0.wait()
            scatter_start(2)
            gb1.wait()
            scatter_start(3)
            return carry
        lax.fori_loop(0, RPT // 4, blk, 0)
        for j in range(4):
            scatter_wait(j)

    @pl.when(cid == 0)
    def _():
        run(yu_hbm, src_hbm, dst_hbm)

    @pl.when(cid == 1)
    def _():
        run(yi_hbm, dst_hbm, src_hbm)

    plsc.subcore_barrier()

    def wout(out_hbm):
        p0 = o0
        for w in _WCH:
            pltpu.sync_copy(acc.at[pl.ds(p0, w)], rows_v.at[pl.ds(0, w)])
            pltpu.sync_copy(rows_v.at[pl.ds(0, w)], out_hbm.at[pl.ds(p0, w)])
            p0 += w

    @pl.when(cid == 0)
    def _():
        wout(si_hbm)

    @pl.when(cid == 1)
    def _():
        wout(su_hbm)


_GRID = 16
_BR = NPAD // _GRID


def _node_spec(width):
    return pl.BlockSpec((_BR, width), lambda i: (i, 0))


def _prep_body(du, di, ut, it, au, ai, yu, yi):
    a_u = lax.rsqrt(jnp.maximum(du[...], 1.0))
    a_i = lax.rsqrt(jnp.maximum(di[...], 1.0))
    au[...] = a_u
    ai[...] = a_i
    yu[...] = ut[...] * a_u
    yi[...] = it[...] * a_i


_prep_call = pl.pallas_call(
    _prep_body,
    grid=(_GRID,),
    in_specs=[_node_spec(1), _node_spec(1), _node_spec(D), _node_spec(D)],
    out_specs=[_node_spec(1), _node_spec(1), _node_spec(D), _node_spec(D)],
    out_shape=[
        jax.ShapeDtypeStruct((NPAD, 1), jnp.float32),
        jax.ShapeDtypeStruct((NPAD, 1), jnp.float32),
        jax.ShapeDtypeStruct((NPAD, D), jnp.float32),
        jax.ShapeDtypeStruct((NPAD, D), jnp.float32),
    ],
)


def _mid_body(au, ai, su, si, yu, yi):
    yu[...] = au[...] * au[...] * su[...]
    yi[...] = ai[...] * ai[...] * si[...]


_mid_call = pl.pallas_call(
    _mid_body,
    grid=(_GRID,),
    in_specs=[_node_spec(1), _node_spec(1), _node_spec(D), _node_spec(D)],
    out_specs=[_node_spec(D), _node_spec(D)],
    out_shape=[
        jax.ShapeDtypeStruct((NPAD, D), jnp.float32),
        jax.ShapeDtypeStruct((NPAD, D), jnp.float32),
    ],
)


def _fin_body(ut, au, su1, su2, it, ai, si1, si2, eu, ei):
    third = jnp.float32(1.0 / 3.0)
    eu[...] = (ut[...] + au[...] * (su1[...] + su2[...])) * third
    ei[...] = (it[...] + ai[...] * (si1[...] + si2[...])) * third


_fin_call = pl.pallas_call(
    _fin_body,
    grid=(_GRID,),
    in_specs=[
        _node_spec(D), _node_spec(1), _node_spec(D), _node_spec(D),
        _node_spec(D), _node_spec(1), _node_spec(D), _node_spec(D),
    ],
    out_specs=[_node_spec(D), _node_spec(D)],
    out_shape=[
        jax.ShapeDtypeStruct((NPAD, D), jnp.float32),
        jax.ShapeDtypeStruct((NPAD, D), jnp.float32),
    ],
)


def kernel(user_table, item_table, user_ids, item_ids, edge_index):
    f32 = jnp.float32
    src = edge_index[0]
    dst = edge_index[1]
    pad_idx = jnp.full((EPAD - E,), N, dtype=jnp.int32)
    src2 = jnp.concatenate([src, pad_idx]).reshape(ROWS, 128)
    dst2 = jnp.concatenate([dst, pad_idx]).reshape(ROWS, 128)

    zpad = jnp.zeros((NPAD - N, D), dtype=f32)
    utab = jnp.concatenate([user_table, zpad], axis=0)
    itab = jnp.concatenate([item_table, zpad], axis=0)

    z1 = jnp.zeros((NPAD,), dtype=f32)
    z2 = jnp.zeros((NPAD, D), dtype=f32)
    ones = jnp.ones((128,), dtype=f32)

    deg_u, deg_i = _deg_kernel(src2, dst2, z1, ones)
    a_u, a_i, yu0, yi0 = _prep_call(
        deg_u.reshape(NPAD, 1), deg_i.reshape(NPAD, 1), utab, itab
    )
    s_i1, s_u1 = _layer_kernel(yu0, yi0, src2, dst2, z2)
    yu1, yi1 = _mid_call(a_u, a_i, s_u1, s_i1)
    s_i2, s_u2 = _layer_kernel(yu1, yi1, src2, dst2, z2)
    emb_u, emb_i = _fin_call(utab, a_u, s_u1, s_u2, itab, a_i, s_i1, s_i2)
    return jnp.concatenate([emb_u[:N], emb_i[:N]], axis=0)


# speedup vs baseline: 1.0823x; 1.0823x over previous
import functools

import jax
import jax.numpy as jnp
from jax import lax
from jax.experimental import pallas as pl
from jax.experimental.pallas import tpu as pltpu
from jax.experimental.pallas import tpu_sc as plsc

N = 50000
D = 32
E = 1600000

TILES = 16
NPAD = 50048
RSLICE = NPAD // TILES
ABUF = 3152

RPT = 784
ROWS = TILES * RPT
EPAD = ROWS * 128

_WCH = (512, 512, 512, 512, 512, 512, 56)
_TCH = (256,) * 12 + (56,)

_MESH = plsc.VectorSubcoreMesh(
    core_axis_name="c", subcore_axis_name="s", num_cores=2, num_subcores=TILES
)
_PARAMS = pltpu.CompilerParams(use_tc_tiling_on_sc=False)

_F32 = jnp.float32
_NODE = jax.ShapeDtypeStruct((NPAD, D), _F32)
_SCAL = jax.ShapeDtypeStruct((NPAD,), _F32)


def _rsqrt_inplace(buf):
    def body(i, c):
        v = jnp.maximum(buf[pl.ds(i * 16, 16)], 1.0)
        bits = lax.bitcast_convert_type(v, jnp.int32)
        seed = jnp.int32(0x5F3759DF) - lax.shift_right_logical(bits, 1)
        y = lax.bitcast_convert_type(seed, _F32)
        for _ in range(3):
            y = y * (1.5 - 0.5 * v * y * y)
        buf[pl.ds(i * 16, 16)] = y
        return c
    lax.fori_loop(0, ABUF // 16, body, 0)


def _square_inplace(buf):
    def body(i, c):
        v = buf[pl.ds(i * 16, 16)]
        buf[pl.ds(i * 16, 16)] = v * v
        return c
    lax.fori_loop(0, ABUF // 16, body, 0)


_GDN = lax.GatherDimensionNumbers(
    offset_dims=(), collapsed_slice_dims=(0,), start_index_map=(0,)
)


def _bcast_lane(v, lane):
    idx = jnp.full((16, 1), lane, jnp.int32)
    return lax.gather(v, idx, _GDN, slice_sizes=(1,),
                      mode=lax.GatherScatterMode.PROMISE_IN_BOUNDS)


def _scale_rows(rows_v, a_buf, w, c0):
    def grp(lanes):
        def body(g, c):
            v = a_buf[pl.ds(c0 + g * 16, 16)]
            for lane in range(lanes):
                av = _bcast_lane(v, lane)
                r = g * 16 + lane
                rows_v[r, pl.ds(0, 16)] = rows_v[r, pl.ds(0, 16)] * av
                rows_v[r, pl.ds(16, 16)] = rows_v[r, pl.ds(16, 16)] * av
            return c
        return body
    lax.fori_loop(0, w // 16, grp(16), 0)
    if w % 16:
        lax.fori_loop(w // 16, w // 16 + 1, grp(w % 16), 0)


def _add_rows(rows_v, other_v, w, scale):
    s = _F32(scale)

    def body(r, c):
        rows_v[r, pl.ds(0, 16)] = (rows_v[r, pl.ds(0, 16)]
                                   + other_v[r, pl.ds(0, 16)]) * s
        rows_v[r, pl.ds(16, 16)] = (rows_v[r, pl.ds(16, 16)]
                                    + other_v[r, pl.ds(16, 16)]) * s
        return c
    lax.fori_loop(0, w, body, 0)


def _zero_acc(zeros2_hbm, rows_v, acc, o0):
    q0 = o0
    for w in _WCH:
        pltpu.sync_copy(zeros2_hbm.at[pl.ds(q0, w)], rows_v.at[pl.ds(0, w)])
        pltpu.sync_copy(rows_v.at[pl.ds(0, w)], acc.at[pl.ds(q0, w)])
        q0 += w


def _deg_pass(idx_hbm, idx_v, ones_v, accdeg, sid, dsem):
    def s_start(j):
        pltpu.async_copy(ones_v, accdeg.at[idx_v.at[j]], dsem, add=True)

    def s_wait(j):
        pltpu.make_async_copy(ones_v, accdeg.at[idx_v.at[j]], dsem).wait()

    def body(b, c):
        @pl.when(b > 0)
        def _():
            for j in range(4):
                s_wait(j)
        r0 = sid * RPT + b * 4
        pltpu.sync_copy(idx_hbm.at[pl.ds(r0, 4)], idx_v)
        for j in range(4):
            s_start(j)
        return c
    lax.fori_loop(0, RPT // 4, body, 0)
    for j in range(4):
        s_wait(j)


def _layer_pass(tab_hbm, g_hbm, s_hbm, gidx_v, sidx_v, rows_v, acc, sid,
                gsem_a, gsem_b, ssem_a, ssem_b):
    ssem = (ssem_a, ssem_a, ssem_b, ssem_b)

    def scatter_start(j):
        pltpu.async_copy(rows_v.at[pl.ds(j * 128, 128)],
                         acc.at[sidx_v.at[j]], ssem[j], add=True)

    def scatter_wait(j):
        pltpu.make_async_copy(rows_v.at[pl.ds(j * 128, 128)],
                              acc.at[sidx_v.at[j]], ssem[j]).wait()

    def blk(b, carry):
        r0 = sid * RPT + b * 4
        @pl.when(b > 0)
        def _():
            scatter_wait(0)
            scatter_wait(1)
        pltpu.sync_copy(g_hbm.at[pl.ds(r0, 2)], gidx_v.at[pl.ds(0, 2)])
        pltpu.sync_copy(s_hbm.at[pl.ds(r0, 2)], sidx_v.at[pl.ds(0, 2)])
        ga0 = pltpu.async_copy(tab_hbm.at[gidx_v.at[0]],
                               rows_v.at[pl.ds(0, 128)], gsem_a)
        ga1 = pltpu.async_copy(tab_hbm.at[gidx_v.at[1]],
                               rows_v.at[pl.ds(128, 128)], gsem_a)
        @pl.when(b > 0)
        def _():
            scatter_wait(2)
            scatter_wait(3)
        pltpu.sync_copy(g_hbm.at[pl.ds(r0 + 2, 2)], gidx_v.at[pl.ds(2, 2)])
        pltpu.sync_copy(s_hbm.at[pl.ds(r0 + 2, 2)], sidx_v.at[pl.ds(2, 2)])
        gb0 = pltpu.async_copy(tab_hbm.at[gidx_v.at[2]],
                               rows_v.at[pl.ds(256, 128)], gsem_b)
        gb1 = pltpu.async_copy(tab_hbm.at[gidx_v.at[3]],
                               rows_v.at[pl.ds(384, 128)], gsem_b)
        ga0.wait()
        scatter_start(0)
        ga1.wait()
        scatter_start(1)
        gb0.wait()
        scatter_start(2)
        gb1.wait()
        scatter_start(3)
        return carry
    lax.fori_loop(0, RPT // 4, blk, 0)
    for j in range(4):
        scatter_wait(j)


@functools.partial(
    pl.kernel,
    out_type=(
        _NODE,
        _NODE,
        _SCAL,
        _SCAL,
        _NODE,
        _NODE,
    ),
    mesh=_MESH,
    scratch_types=[
        pltpu.VMEM((4, 128), jnp.int32),
        pltpu.VMEM((4, 128), jnp.int32),
        pltpu.VMEM((512, D), _F32),
        pltpu.VMEM((128,), _F32),
        pltpu.VMEM((ABUF,), _F32),
        pltpu.VMEM_SHARED((NPAD, D), _F32),
        pltpu.VMEM_SHARED((NPAD,), _F32),
        pltpu.SemaphoreType.DMA,
        pltpu.SemaphoreType.DMA,
        pltpu.SemaphoreType.DMA,
        pltpu.SemaphoreType.DMA,
        pltpu.SemaphoreType.DMA,
    ],
    compiler_params=_PARAMS,
)
def _fwd1_kernel(utab_hbm, itab_hbm, src_hbm, dst_hbm, zeros1_hbm,
                 zeros2_hbm, ones_hbm,
                 yu0_hbm, yi0_hbm, au_hbm, ai_hbm, si1_hbm, su1_hbm,
                 gidx_v, sidx_v, rows_v, ones_v, a_buf,
                 acc, accdeg, gsem_a, gsem_b, ssem_a, ssem_b, dsem):
    cid = lax.axis_index("c")
    sid = lax.axis_index("s")
    o0 = sid * RSLICE

    pltpu.sync_copy(ones_hbm, ones_v)
    pltpu.sync_copy(zeros1_hbm.at[pl.ds(o0, RSLICE)], a_buf.at[pl.ds(0, RSLICE)])
    pltpu.sync_copy(a_buf.at[pl.ds(0, RSLICE)], accdeg.at[pl.ds(o0, RSLICE)])
    _zero_acc(zeros2_hbm, rows_v, acc, o0)
    plsc.subcore_barrier()

    @pl.when(cid == 0)
    def _():
        _deg_pass(src_hbm, gidx_v, ones_v, accdeg, sid, dsem)

    @pl.when(cid == 1)
    def _():
        _deg_pass(dst_hbm, gidx_v, ones_v, accdeg, sid, dsem)

    plsc.subcore_barrier()

    pltpu.sync_copy(accdeg.at[pl.ds(o0, RSLICE)], a_buf.at[pl.ds(0, RSLICE)])
    _rsqrt_inplace(a_buf)

    def scaled_table_out(tab_hbm, a_hbm, y0_hbm):
        pltpu.sync_copy(a_buf.at[pl.ds(0, RSLICE)], a_hbm.at[pl.ds(o0, RSLICE)])
        c0 = 0
        for w in _WCH:
            pltpu.sync_copy(tab_hbm.at[pl.ds(o0 + c0, w)],
                            rows_v.at[pl.ds(0, w)])
            _scale_rows(rows_v, a_buf, w, c0)
            pltpu.sync_copy(rows_v.at[pl.ds(0, w)],
                            y0_hbm.at[pl.ds(o0 + c0, w)])
            c0 += w

    @pl.when(cid == 0)
    def _():
        scaled_table_out(utab_hbm, au_hbm, yu0_hbm)

    @pl.when(cid == 1)
    def _():
        scaled_table_out(itab_hbm, ai_hbm, yi0_hbm)

    plsc.subcore_barrier()

    @pl.when(cid == 0)
    def _():
        _layer_pass(yu0_hbm, src_hbm, dst_hbm, gidx_v, sidx_v, rows_v, acc,
                    sid, gsem_a, gsem_b, ssem_a, ssem_b)

    @pl.when(cid == 1)
    def _():
        _layer_pass(yi0_hbm, dst_hbm, src_hbm, gidx_v, sidx_v, rows_v, acc,
                    sid, gsem_a, gsem_b, ssem_a, ssem_b)

    plsc.subcore_barrier()

    def raw_out(s_hbm):
        c0 = 0
        for w in _WCH:
            pltpu.sync_copy(acc.at[pl.ds(o0 + c0, w)], rows_v.at[pl.ds(0, w)])
            pltpu.sync_copy(rows_v.at[pl.ds(0, w)],
                            s_hbm.at[pl.ds(o0 + c0, w)])
            c0 += w

    @pl.when(cid == 0)
    def _():
        raw_out(si1_hbm)

    @pl.when(cid == 1)
    def _():
        raw_out(su1_hbm)


@functools.partial(
    pl.kernel,
    out_type=(
        _NODE,
        _NODE,
        _NODE,
        _NODE,
    ),
    mesh=_MESH,
    scratch_types=[
        pltpu.VMEM((4, 128), jnp.int32),
        pltpu.VMEM((4, 128), jnp.int32),
        pltpu.VMEM((512, D), _F32),
        pltpu.VMEM((256, D), _F32),
        pltpu.VMEM((ABUF,), _F32),
        pltpu.VMEM_SHARED((NPAD, D), _F32),
        pltpu.SemaphoreType.DMA,
        pltpu.SemaphoreType.DMA,
        pltpu.SemaphoreType.DMA,
        pltpu.SemaphoreType.DMA,
    ],
    compiler_params=_PARAMS,
)
def _fwd2_kernel(si1_hbm, su1_hbm, au_hbm, ai_hbm, utab_hbm, itab_hbm,
                 src_hbm, dst_hbm, zeros2_hbm,
                 eu_hbm, ei_hbm, yi1_hbm, yu1_hbm,
                 gidx_v, sidx_v, rows_v, buf2, a_buf, acc,
                 gsem_a, gsem_b, ssem_a, ssem_b):
    cid = lax.axis_index("c")
    sid = lax.axis_index("s")
    o0 = sid * RSLICE

    _zero_acc(zeros2_hbm, rows_v, acc, o0)

    def y1_out(a_hbm, s1_hbm, y1_hbm):
        pltpu.sync_copy(a_hbm.at[pl.ds(o0, RSLICE)], a_buf.at[pl.ds(0, RSLICE)])
        _square_inplace(a_buf)
        c0 = 0
        for w in _WCH:
            pltpu.sync_copy(s1_hbm.at[pl.ds(o0 + c0, w)],
                            rows_v.at[pl.ds(0, w)])
            _scale_rows(rows_v, a_buf, w, c0)
            pltpu.sync_copy(rows_v.at[pl.ds(0, w)],
                            y1_hbm.at[pl.ds(o0 + c0, w)])
            c0 += w

    @pl.when(cid == 0)
    def _():
        y1_out(ai_hbm, si1_hbm, yi1_hbm)

    @pl.when(cid == 1)
    def _():
        y1_out(au_hbm, su1_hbm, yu1_hbm)

    plsc.subcore_barrier()

    @pl.when(cid == 0)
    def _():
        _layer_pass(yi1_hbm, dst_hbm, src_hbm, gidx_v, sidx_v, rows_v, acc,
                    sid, gsem_a, gsem_b, ssem_a, ssem_b)

    @pl.when(cid == 1)
    def _():
        _layer_pass(yu1_hbm, src_hbm, dst_hbm, gidx_v, sidx_v, rows_v, acc,
                    sid, gsem_a, gsem_b, ssem_a, ssem_b)

    plsc.subcore_barrier()

    def tail(a_hbm, s1_hbm, tab_hbm, out_hbm):
        pltpu.sync_copy(a_hbm.at[pl.ds(o0, RSLICE)], a_buf.at[pl.ds(0, RSLICE)])
        c0 = 0
        for w in _TCH:
            pltpu.sync_copy(acc.at[pl.ds(o0 + c0, w)], rows_v.at[pl.ds(0, w)])
            pltpu.sync_copy(s1_hbm.at[pl.ds(o0 + c0, w)], buf2.at[pl.ds(0, w)])
            _add_rows(rows_v, buf2, w, 1.0)
            _scale_rows(rows_v, a_buf, w, c0)
            pltpu.sync_copy(tab_hbm.at[pl.ds(o0 + c0, w)], buf2.at[pl.ds(0, w)])
            _add_rows(rows_v, buf2, w, 1.0 / 3.0)
            pltpu.sync_copy(rows_v.at[pl.ds(0, w)],
                            out_hbm.at[pl.ds(o0 + c0, w)])
            c0 += w

    @pl.when(cid == 0)
    def _():
        tail(au_hbm, su1_hbm, utab_hbm, eu_hbm)

    @pl.when(cid == 1)
    def _():
        tail(ai_hbm, si1_hbm, itab_hbm, ei_hbm)


def kernel(user_table, item_table, user_ids, item_ids, edge_index):
    src = edge_index[0]
    dst = edge_index[1]
    pad_idx = jnp.full((EPAD - E,), N, dtype=jnp.int32)
    src2 = jnp.concatenate([src, pad_idx]).reshape(ROWS, 128)
    dst2 = jnp.concatenate([dst, pad_idx]).reshape(ROWS, 128)

    zpad = jnp.zeros((NPAD - N, D), dtype=_F32)
    utab = jnp.concatenate([user_table, zpad], axis=0)
    itab = jnp.concatenate([item_table, zpad], axis=0)
    z1 = jnp.zeros((NPAD,), dtype=_F32)
    z2 = jnp.zeros((NPAD, D), dtype=_F32)
    ones = jnp.ones((128,), dtype=_F32)

    (yu0, yi0, a_u, a_i, s_i1, s_u1) = _fwd1_kernel(
        utab, itab, src2, dst2, z1, z2, ones
    )
    del yu0, yi0
    emb_u, emb_i, yi1, yu1 = _fwd2_kernel(
        s_i1, s_u1, a_u, a_i, utab, itab, src2, dst2, z2
    )
    del yi1, yu1
    return jnp.concatenate([emb_u[:N], emb_i[:N]], axis=0)


# speedup vs baseline: 1.4602x; 1.3492x over previous
import functools

import jax
import jax.numpy as jnp
from jax import lax
from jax.experimental import pallas as pl
from jax.experimental.pallas import tpu as pltpu
from jax.experimental.pallas import tpu_sc as plsc

N = 50000
D = 32
E = 1600000

TILES = 16
NPAD = 50048
RSLICE = NPAD // TILES
ABUF = 3152

RPT = 784
ROWS = TILES * RPT
EPAD = ROWS * 128

_WCH = (512, 512, 512, 512, 512, 512, 56)
_TCH = (256,) * 12 + (56,)

_MESH = plsc.VectorSubcoreMesh(
    core_axis_name="c", subcore_axis_name="s", num_cores=2, num_subcores=TILES
)
_PARAMS = pltpu.CompilerParams(use_tc_tiling_on_sc=False)

_F32 = jnp.float32
_NODE = jax.ShapeDtypeStruct((NPAD, D), _F32)
_SCAL = jax.ShapeDtypeStruct((NPAD,), _F32)


def _rsqrt_inplace(buf):
    def body(i, c):
        v = jnp.maximum(buf[pl.ds(i * 16, 16)], 1.0)
        bits = lax.bitcast_convert_type(v, jnp.int32)
        seed = jnp.int32(0x5F3759DF) - lax.shift_right_logical(bits, 1)
        y = lax.bitcast_convert_type(seed, _F32)
        for _ in range(3):
            y = y * (1.5 - 0.5 * v * y * y)
        buf[pl.ds(i * 16, 16)] = y
        return c
    lax.fori_loop(0, ABUF // 16, body, 0)


def _square_inplace(buf):
    def body(i, c):
        v = buf[pl.ds(i * 16, 16)]
        buf[pl.ds(i * 16, 16)] = v * v
        return c
    lax.fori_loop(0, ABUF // 16, body, 0)


_GDN = lax.GatherDimensionNumbers(
    offset_dims=(), collapsed_slice_dims=(0,), start_index_map=(0,)
)


def _bcast_lane(v, lane):
    idx = jnp.full((16, 1), lane, jnp.int32)
    return lax.gather(v, idx, _GDN, slice_sizes=(1,),
                      mode=lax.GatherScatterMode.PROMISE_IN_BOUNDS)


def _scale_rows(rows_v, a_buf, w, c0):
    def grp(lanes):
        def body(g, c):
            v = a_buf[pl.ds(c0 + g * 16, 16)]
            for lane in range(lanes):
                av = _bcast_lane(v, lane)
                r = g * 16 + lane
                rows_v[r, pl.ds(0, 16)] = rows_v[r, pl.ds(0, 16)] * av
                rows_v[r, pl.ds(16, 16)] = rows_v[r, pl.ds(16, 16)] * av
            return c
        return body
    lax.fori_loop(0, w // 16, grp(16), 0)
    if w % 16:
        lax.fori_loop(w // 16, w // 16 + 1, grp(w % 16), 0)


def _add_rows_upper(rows_v, w, scale):
    s = _F32(scale)

    def body(r, c):
        rows_v[r, pl.ds(0, 16)] = (rows_v[r, pl.ds(0, 16)]
                                   + rows_v[256 + r, pl.ds(0, 16)]) * s
        rows_v[r, pl.ds(16, 16)] = (rows_v[r, pl.ds(16, 16)]
                                    + rows_v[256 + r, pl.ds(16, 16)]) * s
        return c
    lax.fori_loop(0, w, body, 0)


def _zero_acc(zeros2_hbm, rows_v, acc, o0):
    q0 = o0
    for w in _WCH:
        pltpu.sync_copy(zeros2_hbm.at[pl.ds(q0, w)], rows_v.at[pl.ds(0, w)])
        pltpu.sync_copy(rows_v.at[pl.ds(0, w)], acc.at[pl.ds(q0, w)])
        q0 += w


def _publish_flag(flags_hbm, fbuf, row):
    fbuf[pl.ds(0, 16)] = jnp.full((16,), 1, jnp.int32)
    pltpu.sync_copy(fbuf, flags_hbm.at[row])


_SPIN_OUT = 512
_SPIN_IN = 256


def _spin_flag(flags_hbm, fbuf, row, sem):
    fbuf[pl.ds(0, 16)] = jnp.zeros((16,), jnp.int32)

    def outer(i, c):
        v = fbuf[pl.ds(0, 16)]

        @pl.when(v[0] == 0)
        def _():
            def inner(j, c2):
                w = fbuf[pl.ds(0, 16)]

                @pl.when(w[0] == 0)
                def _():
                    pltpu.async_copy(flags_hbm.at[row], fbuf, sem)
                    pltpu.make_async_copy(flags_hbm.at[row], fbuf,
                                          sem).wait()
                return c2
            lax.fori_loop(0, _SPIN_IN, inner, 0)
        return c
    lax.fori_loop(0, _SPIN_OUT, outer, 0)


NBLK = RPT // 4


def _deg_pass(idx_hbm, idx_v, ones_v, accdeg, sid, dsem, isem):
    base = sid * RPT

    def i_start(b, o):
        pltpu.async_copy(idx_hbm.at[pl.ds(base + b * 4, 4)],
                         idx_v.at[pl.ds(o, 4)], isem)

    def i_wait(b, o):
        pltpu.make_async_copy(idx_hbm.at[pl.ds(base + b * 4, 4)],
                              idx_v.at[pl.ds(o, 4)], isem).wait()

    def s_start(o, j):
        pltpu.async_copy(ones_v, accdeg.at[idx_v.at[o + j]], dsem, add=True)

    def s_wait(o, j):
        pltpu.make_async_copy(ones_v, accdeg.at[idx_v.at[o + j]],
                              dsem).wait()

    i_start(0, 0)

    def body(b, c):
        o = lax.bitwise_and(b, 1) * 4
        po = 4 - o
        @pl.when(b > 0)
        def _():
            for j in range(4):
                s_wait(po, j)
        @pl.when(b + 1 < NBLK)
        def _():
            i_start(b + 1, po)
        i_wait(b, o)
        for j in range(4):
            s_start(o, j)
        return c
    lax.fori_loop(0, NBLK, body, 0)
    for j in range(4):
        s_wait(((NBLK - 1) % 2) * 4, j)


def _layer_pass(tab_hbm, g_hbm, s_hbm, gidx_v, sidx_v, rows_v, acc, sid,
                gsem_a, gsem_b, ssem_a, ssem_b, isem):
    gsem = (gsem_a, gsem_a, gsem_b, gsem_b)
    ssem = (ssem_a, ssem_a, ssem_b, ssem_b)
    base = sid * RPT

    def i_start(b, o):
        pltpu.async_copy(g_hbm.at[pl.ds(base + b * 4, 4)],
                         gidx_v.at[pl.ds(o, 4)], isem)
        pltpu.async_copy(s_hbm.at[pl.ds(base + b * 4, 4)],
                         sidx_v.at[pl.ds(o, 4)], isem)

    def i_wait(b, o):
        pltpu.make_async_copy(g_hbm.at[pl.ds(base + b * 4, 4)],
                              gidx_v.at[pl.ds(o, 4)], isem).wait()
        pltpu.make_async_copy(s_hbm.at[pl.ds(base + b * 4, 4)],
                              sidx_v.at[pl.ds(o, 4)], isem).wait()

    def scatter_start(o, j):
        pltpu.async_copy(rows_v.at[pl.ds(j * 128, 128)],
                         acc.at[sidx_v.at[o + j]], ssem[j], add=True)

    def scatter_wait(o, j):
        pltpu.make_async_copy(rows_v.at[pl.ds(j * 128, 128)],
                              acc.at[sidx_v.at[o + j]], ssem[j]).wait()

    i_start(0, 0)

    def blk(b, carry):
        o = lax.bitwise_and(b, 1) * 4
        po = 4 - o
        @pl.when(b > 0)
        def _():
            for j in range(4):
                scatter_wait(po, j)
        @pl.when(b + 1 < NBLK)
        def _():
            i_start(b + 1, po)
        i_wait(b, o)
        gd = [pltpu.async_copy(tab_hbm.at[gidx_v.at[o + j]],
                               rows_v.at[pl.ds(j * 128, 128)], gsem[j])
              for j in range(4)]
        for j in range(4):
            gd[j].wait()
            scatter_start(o, j)
        return carry
    lax.fori_loop(0, NBLK, blk, 0)
    for j in range(4):
        scatter_wait(((NBLK - 1) % 2) * 4, j)


@functools.partial(
    pl.kernel,
    out_type=(
        _NODE,
        _NODE,
        _NODE,
        _NODE,
        _SCAL,
        _SCAL,
        _NODE,
        _NODE,
        _NODE,
        _NODE,
    ),
    mesh=_MESH,
    scratch_types=[
        pltpu.VMEM((8, 128), jnp.int32),
        pltpu.VMEM((8, 128), jnp.int32),
        pltpu.VMEM((512, D), _F32),
        pltpu.VMEM((128,), _F32),
        pltpu.VMEM((ABUF,), _F32),
        pltpu.VMEM((16,), jnp.int32),
        pltpu.VMEM_SHARED((NPAD, D), _F32),
        pltpu.VMEM_SHARED((NPAD,), _F32),
        pltpu.SemaphoreType.DMA,
        pltpu.SemaphoreType.DMA,
        pltpu.SemaphoreType.DMA,
        pltpu.SemaphoreType.DMA,
        pltpu.SemaphoreType.DMA,
        pltpu.SemaphoreType.DMA,
    ],
    compiler_params=_PARAMS,
)
def _fwd_kernel(utab_hbm, itab_hbm, src_hbm, dst_hbm, zeros1_hbm,
                zeros2_hbm, ones_hbm, flags_hbm,
                eu_hbm, ei_hbm, yu0_hbm, yi0_hbm, au_hbm, ai_hbm,
                si1_hbm, su1_hbm, yi1_hbm, yu1_hbm,
                gidx_v, sidx_v, rows_v, ones_v, a_buf, fbuf,
                acc, accdeg, gsem_a, gsem_b, ssem_a, ssem_b, dsem, isem):
    cid = lax.axis_index("c")
    sid = lax.axis_index("s")
    o0 = sid * RSLICE

    pltpu.sync_copy(ones_hbm, ones_v)
    pltpu.sync_copy(zeros1_hbm.at[pl.ds(o0, RSLICE)], a_buf.at[pl.ds(0, RSLICE)])
    pltpu.sync_copy(a_buf.at[pl.ds(0, RSLICE)], accdeg.at[pl.ds(o0, RSLICE)])
    _zero_acc(zeros2_hbm, rows_v, acc, o0)
    plsc.subcore_barrier()

    @pl.when(cid == 0)
    def _():
        _deg_pass(src_hbm, gidx_v, ones_v, accdeg, sid, dsem, isem)

    @pl.when(cid == 1)
    def _():
        _deg_pass(dst_hbm, gidx_v, ones_v, accdeg, sid, dsem, isem)

    plsc.subcore_barrier()

    pltpu.sync_copy(accdeg.at[pl.ds(o0, RSLICE)], a_buf.at[pl.ds(0, RSLICE)])
    _rsqrt_inplace(a_buf)

    def scaled_table_out(tab_hbm, a_hbm, y0_hbm):
        pltpu.sync_copy(a_buf.at[pl.ds(0, RSLICE)], a_hbm.at[pl.ds(o0, RSLICE)])
        c0 = 0
        for w in _WCH:
            pltpu.sync_copy(tab_hbm.at[pl.ds(o0 + c0, w)],
                            rows_v.at[pl.ds(0, w)])
            _scale_rows(rows_v, a_buf, w, c0)
            pltpu.sync_copy(rows_v.at[pl.ds(0, w)],
                            y0_hbm.at[pl.ds(o0 + c0, w)])
            c0 += w

    @pl.when(cid == 0)
    def _():
        scaled_table_out(utab_hbm, au_hbm, yu0_hbm)

    @pl.when(cid == 1)
    def _():
        scaled_table_out(itab_hbm, ai_hbm, yi0_hbm)

    plsc.subcore_barrier()

    @pl.when((cid == 0) & (sid == 0))
    def _():
        _publish_flag(flags_hbm, fbuf, 0)

    @pl.when((cid == 1) & (sid == 0))
    def _():
        _publish_flag(flags_hbm, fbuf, 2)

    @pl.when(cid == 0)
    def _():
        _layer_pass(yu0_hbm, src_hbm, dst_hbm, gidx_v, sidx_v, rows_v, acc,
                    sid, gsem_a, gsem_b, ssem_a, ssem_b, isem)

    @pl.when(cid == 1)
    def _():
        _layer_pass(yi0_hbm, dst_hbm, src_hbm, gidx_v, sidx_v, rows_v, acc,
                    sid, gsem_a, gsem_b, ssem_a, ssem_b, isem)

    plsc.subcore_barrier()

    def raw_out(s_hbm):
        c0 = 0
        for w in _WCH:
            pltpu.sync_copy(acc.at[pl.ds(o0 + c0, w)], rows_v.at[pl.ds(0, w)])
            pltpu.sync_copy(rows_v.at[pl.ds(0, w)],
                            s_hbm.at[pl.ds(o0 + c0, w)])
            c0 += w

    @pl.when(cid == 0)
    def _():
        raw_out(si1_hbm)

    @pl.when(cid == 1)
    def _():
        raw_out(su1_hbm)

    plsc.subcore_barrier()

    @pl.when((cid == 0) & (sid == 0))
    def _():
        _publish_flag(flags_hbm, fbuf, 1)

    @pl.when((cid == 1) & (sid == 0))
    def _():
        _publish_flag(flags_hbm, fbuf, 3)

    _zero_acc(zeros2_hbm, rows_v, acc, o0)

    def y1_out(a_hbm, s1_hbm, y1_hbm):
        pltpu.sync_copy(a_hbm.at[pl.ds(o0, RSLICE)], a_buf.at[pl.ds(0, RSLICE)])
        _square_inplace(a_buf)
        c0 = 0
        for w in _WCH:
            pltpu.sync_copy(s1_hbm.at[pl.ds(o0 + c0, w)],
                            rows_v.at[pl.ds(0, w)])
            _scale_rows(rows_v, a_buf, w, c0)
            pltpu.sync_copy(rows_v.at[pl.ds(0, w)],
                            y1_hbm.at[pl.ds(o0 + c0, w)])
            c0 += w

    @pl.when(cid == 0)
    def _():
        _spin_flag(flags_hbm, fbuf, 2, dsem)
        y1_out(ai_hbm, si1_hbm, yi1_hbm)

    @pl.when(cid == 1)
    def _():
        _spin_flag(flags_hbm, fbuf, 0, dsem)
        y1_out(au_hbm, su1_hbm, yu1_hbm)

    plsc.subcore_barrier()

    @pl.when(cid == 0)
    def _():
        _layer_pass(yi1_hbm, dst_hbm, src_hbm, gidx_v, sidx_v, rows_v, acc,
                    sid, gsem_a, gsem_b, ssem_a, ssem_b, isem)

    @pl.when(cid == 1)
    def _():
        _layer_pass(yu1_hbm, src_hbm, dst_hbm, gidx_v, sidx_v, rows_v, acc,
                    sid, gsem_a, gsem_b, ssem_a, ssem_b, isem)

    plsc.subcore_barrier()

    def tail(a_hbm, s1_hbm, tab_hbm, out_hbm):
        pltpu.sync_copy(a_hbm.at[pl.ds(o0, RSLICE)], a_buf.at[pl.ds(0, RSLICE)])
        c0 = 0
        for w in _TCH:
            pltpu.sync_copy(acc.at[pl.ds(o0 + c0, w)], rows_v.at[pl.ds(0, w)])
            pltpu.sync_copy(s1_hbm.at[pl.ds(o0 + c0, w)],
                            rows_v.at[pl.ds(256, w)])
            _add_rows_upper(rows_v, w, 1.0)
            _scale_rows(rows_v, a_buf, w, c0)
            pltpu.sync_copy(tab_hbm.at[pl.ds(o0 + c0, w)],
                            rows_v.at[pl.ds(256, w)])
            _add_rows_upper(rows_v, w, 1.0 / 3.0)
            pltpu.sync_copy(rows_v.at[pl.ds(0, w)],
                            out_hbm.at[pl.ds(o0 + c0, w)])
            c0 += w

    @pl.when(cid == 0)
    def _():
        _spin_flag(flags_hbm, fbuf, 3, dsem)
        tail(au_hbm, su1_hbm, utab_hbm, eu_hbm)

    @pl.when(cid == 1)
    def _():
        _spin_flag(flags_hbm, fbuf, 1, dsem)
        tail(ai_hbm, si1_hbm, itab_hbm, ei_hbm)


def kernel(user_table, item_table, user_ids, item_ids, edge_index):
    src = edge_index[0]
    dst = edge_index[1]
    pad_idx = jnp.full((EPAD - E,), N, dtype=jnp.int32)
    src2 = jnp.concatenate([src, pad_idx]).reshape(ROWS, 128)
    dst2 = jnp.concatenate([dst, pad_idx]).reshape(ROWS, 128)

    zpad = jnp.zeros((NPAD - N, D), dtype=_F32)
    utab = jnp.concatenate([user_table, zpad], axis=0)
    itab = jnp.concatenate([item_table, zpad], axis=0)
    z1 = jnp.zeros((NPAD,), dtype=_F32)
    z2 = jnp.zeros((NPAD, D), dtype=_F32)
    ones = jnp.ones((128,), dtype=_F32)
    flags = lax.shift_right_arithmetic(src[:64], 31).reshape(4, 16)

    outs = _fwd_kernel(utab, itab, src2, dst2, z1, z2, ones, flags)
    emb_u, emb_i = outs[0], outs[1]
    return jnp.concatenate([emb_u[:N], emb_i[:N]], axis=0)
